# Initial kernel scaffold; baseline (speedup 1.0000x reference)
import functools

import jax
import jax.numpy as jnp
from jax.experimental import pallas as pl
from jax.experimental.pallas import tpu as pltpu

N = 10000
E = 320000
D = 128
OUT = 64

ROWS = 1250
GRID = N // ROWS


def _ln(x, g, b):
    mu = jnp.mean(x, axis=-1, keepdims=True)
    var = jnp.mean((x - mu) ** 2, axis=-1, keepdims=True)
    return (x - mu) * jax.lax.rsqrt(var + 1e-5) * g + b


def _row_spec(cols):
    return pl.BlockSpec((ROWS, cols), lambda i: (i, 0))


def _full_spec(shape):
    nd = len(shape)
    return pl.BlockSpec(shape, lambda i: (0,) * nd)


def _dense_call(body, out_cols_list, *args):
    in_specs = []
    for a, rowb in args:
        in_specs.append(_row_spec(a.shape[-1]) if rowb else _full_spec(a.shape))
    outs = [jax.ShapeDtypeStruct((N, c), jnp.float32) for c in out_cols_list]
    res = pl.pallas_call(
        body,
        grid=(GRID,),
        in_specs=in_specs,
        out_specs=[_row_spec(c) for c in out_cols_list],
        out_shape=outs,
    )(*[a for a, _ in args])
    return res


def _in_body(x_ref, w_ref, b_ref, g_ref, beta_ref, o_ref):
    h = jnp.dot(x_ref[...], w_ref[...], preferred_element_type=jnp.float32)
    h = h + b_ref[...]
    o_ref[...] = jax.nn.relu(_ln(h, g_ref[...], beta_ref[...]))


def _ab_body(h_ref, wt_ref, wb_ref, mb_ref, g_ref, beta_ref,
             a_ref, b_ref, sl_ref):
    h = h_ref[...]
    a = jnp.dot(h, wt_ref[...], preferred_element_type=jnp.float32)
    b = jnp.dot(h, wb_ref[...], preferred_element_type=jnp.float32)
    b = b + mb_ref[...]
    a_ref[...] = a
    b_ref[...] = b
    sl_ref[...] = jax.nn.relu(_ln(a + b, g_ref[...], beta_ref[...]))


def _upd_body(residual, psum_ref, sl_ref, cnt_ref, h_ref,
              ut_ref, ub_ref, bias_ref, g_ref, beta_ref, o_ref):
    h = h_ref[...]
    aggr = (psum_ref[...] + sl_ref[...]) / cnt_ref[...]
    u = jnp.dot(aggr, ut_ref[...], preferred_element_type=jnp.float32)
    u = u + jnp.dot(h, ub_ref[...], preferred_element_type=jnp.float32)
    u = u + bias_ref[...]
    hn = jax.nn.relu(_ln(u, g_ref[...], beta_ref[...]))
    if residual:
        hn = 0.5 * (hn + h)
    o_ref[...] = hn


def _out_body(h_ref, w1_ref, b1_ref, g_ref, beta_ref, w2_ref, b2_ref, o_ref):
    o = jnp.dot(h_ref[...], w1_ref[...], preferred_element_type=jnp.float32)
    o = jax.nn.relu(_ln(o + b1_ref[...], g_ref[...], beta_ref[...]))
    o = jnp.dot(o, w2_ref[...], preferred_element_type=jnp.float32) + b2_ref[...]
    m = jnp.max(o, axis=-1, keepdims=True)
    lse = jnp.log(jnp.sum(jnp.exp(o - m), axis=-1, keepdims=True)) + m
    o_ref[...] = o - lse


@functools.cache
def _upd_body_i(residual):
    return functools.partial(_upd_body, residual)


def kernel(x, edge_index, params):
    p = params
    h = _dense_call(_in_body, [D],
                    (x, True), (p['in_W'], False), (p['in_b'], False),
                    (p['in_g'], False), (p['in_beta'], False))

    src = edge_index[0]
    dst = edge_index[1]
    ones = jnp.ones((E,), jnp.float32)
    cnt = jax.ops.segment_sum(ones, dst, num_segments=N) + 1.0
    cnt = cnt[:, None]

    for i, lp in enumerate(p['layers']):
        wt = lp['msg_W'][:D]
        wb = lp['msg_W'][D:]
        a, b, sl = _dense_call(
            _ab_body, [D, D, D],
            (h, True), (wt, False), (wb, False), (lp['msg_b'], False),
            (lp['msg_g'], False), (lp['msg_beta'], False))

        v = a[dst] + b[src]
        mu = jnp.mean(v, axis=-1, keepdims=True)
        var = jnp.mean((v - mu) ** 2, axis=-1, keepdims=True)
        m = jax.nn.relu((v - mu) * jax.lax.rsqrt(var + 1e-5)
                        * lp['msg_g'] + lp['msg_beta'])
        psum = jax.ops.segment_sum(m, dst, num_segments=N)

        h = _dense_call(
            _upd_body_i(i > 0), [D],
            (psum, True), (sl, True), (cnt, True), (h, True),
            (lp['upd_W'][:D], False), (lp['upd_W'][D:], False),
            (lp['upd_b'], False), (lp['upd_g'], False), (lp['upd_beta'], False))

    o = _dense_call(_out_body, [OUT],
                    (h, True), (p['out_W1'], False), (p['out_b1'], False),
                    (p['out_g'], False), (p['out_beta'], False),
                    (p['out_W2'], False), (p['out_b2'], False))
    return o


# speedup vs baseline: 1.3605x; 1.3605x over previous
import functools

import jax
import jax.numpy as jnp
from jax.experimental import pallas as pl
from jax.experimental.pallas import tpu as pltpu

N = 10000
E = 320000
D = 128
OUT = 64

ROWS = 1000
GRID = N // ROWS


def _ln(x, g, b):
    mu = jnp.mean(x, axis=-1, keepdims=True)
    var = jnp.mean((x - mu) ** 2, axis=-1, keepdims=True)
    return (x - mu) * jax.lax.rsqrt(var + 1e-5) * g + b


def _row_spec(cols):
    return pl.BlockSpec((ROWS, cols), lambda i: (i, 0))


def _full_spec(shape):
    nd = len(shape)
    return pl.BlockSpec(shape, lambda i: (0,) * nd)


def _dense_call(body, out_cols_list, *args):
    in_specs = []
    for a, rowb in args:
        in_specs.append(_row_spec(a.shape[-1]) if rowb else _full_spec(a.shape))
    outs = [jax.ShapeDtypeStruct((N, c), jnp.float32) for c in out_cols_list]
    res = pl.pallas_call(
        body,
        grid=(GRID,),
        in_specs=in_specs,
        out_specs=[_row_spec(c) for c in out_cols_list],
        out_shape=outs,
    )(*[a for a, _ in args])
    return res[0] if len(out_cols_list) == 1 else res


def _in_body(x_ref, w_ref, b_ref, g_ref, beta_ref, o_ref):
    h = jnp.dot(x_ref[...], w_ref[...], preferred_element_type=jnp.float32)
    h = h + b_ref[...]
    o_ref[...] = jax.nn.relu(_ln(h, g_ref[...], beta_ref[...]))


def _ab_body(h_ref, wt_ref, wb_ref, mb_ref, g_ref, beta_ref,
             a_ref, b_ref, sl_ref):
    h = h_ref[...]
    a = jnp.dot(h, wt_ref[...], preferred_element_type=jnp.float32)
    b = jnp.dot(h, wb_ref[...], preferred_element_type=jnp.float32)
    b = b + mb_ref[...]
    a_ref[...] = a
    b_ref[...] = b
    sl_ref[...] = jax.nn.relu(_ln(a + b, g_ref[...], beta_ref[...]))


def _upd_body(residual, psum_ref, sl_ref, cnt_ref, h_ref,
              ut_ref, ub_ref, bias_ref, g_ref, beta_ref, o_ref):
    h = h_ref[...]
    aggr = (psum_ref[...] + sl_ref[...]) / cnt_ref[...]
    u = jnp.dot(aggr, ut_ref[...], preferred_element_type=jnp.float32)
    u = u + jnp.dot(h, ub_ref[...], preferred_element_type=jnp.float32)
    u = u + bias_ref[...]
    hn = jax.nn.relu(_ln(u, g_ref[...], beta_ref[...]))
    if residual:
        hn = 0.5 * (hn + h)
    o_ref[...] = hn


def _out_body(h_ref, w1_ref, b1_ref, g_ref, beta_ref, w2_ref, b2_ref, o_ref):
    o = jnp.dot(h_ref[...], w1_ref[...], preferred_element_type=jnp.float32)
    o = jax.nn.relu(_ln(o + b1_ref[...], g_ref[...], beta_ref[...]))
    o = jnp.dot(o, w2_ref[...], preferred_element_type=jnp.float32) + b2_ref[...]
    m = jnp.max(o, axis=-1, keepdims=True)
    lse = jnp.log(jnp.sum(jnp.exp(o - m), axis=-1, keepdims=True)) + m
    o_ref[...] = o - lse


@functools.cache
def _upd_body_i(residual):
    return functools.partial(_upd_body, residual)


def kernel(x, edge_index, params):
    p = params
    h = _dense_call(_in_body, [D],
                    (x, True), (p['in_W'], False), (p['in_b'], False),
                    (p['in_g'], False), (p['in_beta'], False))

    src = edge_index[0]
    dst = edge_index[1]
    ones = jnp.ones((E,), jnp.float32)
    cnt = jax.ops.segment_sum(ones, dst, num_segments=N) + 1.0
    cnt = cnt[:, None]

    for i, lp in enumerate(p['layers']):
        wt = lp['msg_W'][:D]
        wb = lp['msg_W'][D:]
        a, b, sl = _dense_call(
            _ab_body, [D, D, D],
            (h, True), (wt, False), (wb, False), (lp['msg_b'], False),
            (lp['msg_g'], False), (lp['msg_beta'], False))

        v = a[dst] + b[src]
        mu = jnp.mean(v, axis=-1, keepdims=True)
        var = jnp.mean((v - mu) ** 2, axis=-1, keepdims=True)
        m = jax.nn.relu((v - mu) * jax.lax.rsqrt(var + 1e-5)
                        * lp['msg_g'] + lp['msg_beta'])
        psum = jax.ops.segment_sum(m, dst, num_segments=N)

        h = _dense_call(
            _upd_body_i(i > 0), [D],
            (psum, True), (sl, True), (cnt, True), (h, True),
            (lp['upd_W'][:D], False), (lp['upd_W'][D:], False),
            (lp['upd_b'], False), (lp['upd_g'], False), (lp['upd_beta'], False))

    o = _dense_call(_out_body, [OUT],
                    (h, True), (p['out_W1'], False), (p['out_b1'], False),
                    (p['out_g'], False), (p['out_beta'], False),
                    (p['out_W2'], False), (p['out_b2'], False))
    return o


# speedup vs baseline: 4.1687x; 3.0641x over previous
import functools

import jax
import jax.numpy as jnp
from jax.experimental import pallas as pl
from jax.experimental.pallas import tpu as pltpu
from jax.experimental.pallas import tpu_sc as plsc

N = 10000
E = 320000
D = 128
OUT = 64

ROWS = 1000
GRID = N // ROWS

NC = 2
NS = 16
LANES = 16

EPW = E // (NC * NS)
CHUNK = 80
NCH = EPW // CHUNK
NPAD = 10112
RPS = NPAD // NS
ZR = 64


def _ln(x, g, b):
    mu = jnp.mean(x, axis=-1, keepdims=True)
    var = jnp.mean((x - mu) ** 2, axis=-1, keepdims=True)
    return (x - mu) * jax.lax.rsqrt(var + 1e-5) * g + b


def _row_spec(cols):
    return pl.BlockSpec((ROWS, cols), lambda i: (i, 0))


def _row3_spec(cols):
    return pl.BlockSpec((NC, ROWS, cols), lambda i: (0, i, 0))


def _full_spec(shape):
    nd = len(shape)
    return pl.BlockSpec(shape, lambda i: (0,) * nd)


def _dense_call(body, out_cols_list, *args):
    in_specs = []
    for a, kind in args:
        if kind == 'row':
            in_specs.append(_row_spec(a.shape[-1]))
        elif kind == 'row3':
            in_specs.append(_row3_spec(a.shape[-1]))
        else:
            in_specs.append(_full_spec(a.shape))
    outs = [jax.ShapeDtypeStruct((N, c), jnp.float32) for c in out_cols_list]
    res = pl.pallas_call(
        body,
        grid=(GRID,),
        in_specs=in_specs,
        out_specs=[_row_spec(c) for c in out_cols_list],
        out_shape=outs,
    )(*[a for a, _ in args])
    return res[0] if len(out_cols_list) == 1 else res


def _in_body(x_ref, w_ref, b_ref, g_ref, beta_ref, o_ref):
    h = jnp.dot(x_ref[...], w_ref[...], preferred_element_type=jnp.float32)
    h = h + b_ref[...]
    o_ref[...] = jax.nn.relu(_ln(h, g_ref[...], beta_ref[...]))


def _ab_body(h_ref, wt_ref, wb_ref, mb_ref, g_ref, beta_ref,
             a_ref, b_ref, sl_ref):
    h = h_ref[...]
    a = jnp.dot(h, wt_ref[...], preferred_element_type=jnp.float32)
    b = jnp.dot(h, wb_ref[...], preferred_element_type=jnp.float32)
    b = b + mb_ref[...]
    a_ref[...] = a
    b_ref[...] = b
    sl_ref[...] = jax.nn.relu(_ln(a + b, g_ref[...], beta_ref[...]))


def _upd_body(residual, p_ref, sl_ref, c_ref, h_ref,
              ut_ref, ub_ref, bias_ref, g_ref, beta_ref, o_ref):
    h = h_ref[...]
    cnt = c_ref[0, :, :1] + c_ref[1, :, :1] + 1.0
    aggr = (p_ref[0] + p_ref[1] + sl_ref[...]) / cnt
    u = jnp.dot(aggr, ut_ref[...], preferred_element_type=jnp.float32)
    u = u + jnp.dot(h, ub_ref[...], preferred_element_type=jnp.float32)
    u = u + bias_ref[...]
    hn = jax.nn.relu(_ln(u, g_ref[...], beta_ref[...]))
    if residual:
        hn = 0.5 * (hn + h)
    o_ref[...] = hn


def _out_body(h_ref, w1_ref, b1_ref, g_ref, beta_ref, w2_ref, b2_ref, o_ref):
    o = jnp.dot(h_ref[...], w1_ref[...], preferred_element_type=jnp.float32)
    o = jax.nn.relu(_ln(o + b1_ref[...], g_ref[...], beta_ref[...]))
    o = jnp.dot(o, w2_ref[...], preferred_element_type=jnp.float32) + b2_ref[...]
    m = jnp.max(o, axis=-1, keepdims=True)
    lse = jnp.log(jnp.sum(jnp.exp(o - m), axis=-1, keepdims=True)) + m
    o_ref[...] = o - lse


@functools.cache
def _upd_body_i(residual):
    return functools.partial(_upd_body, residual)


def _lane_sum(v):
    dnums = jax.lax.GatherDimensionNumbers(
        offset_dims=(), collapsed_slice_dims=(0,), start_index_map=(0,))
    for k in (8, 4, 2, 1):
        idx = jax.lax.iota(jnp.int32, LANES) ^ k
        v = v + jax.lax.gather(
            v, idx[:, None], dnums, slice_sizes=(1,),
            mode=jax.lax.GatherScatterMode.PROMISE_IN_BOUNDS)
    return v

def _zero_acc_slice(zsrc, acc, row0):
    for k in range(RPS // ZR):
        pltpu.sync_copy(zsrc.at[pl.ds(0, ZR)],
                        acc.at[pl.ds(row0 + k * ZR, ZR)])
    pltpu.sync_copy(zsrc.at[pl.ds(0, RPS % ZR)],
                    acc.at[pl.ds(row0 + (RPS // ZR) * ZR, RPS % ZR)])


def _edge_body(a_hbm, b_hbm, src_hbm, dst_hbm, p_hbm,
               sidx, didx, arow, brow, acc, sem0, sem1):
    cid = jax.lax.axis_index("c")
    sid = jax.lax.axis_index("s")

    @pl.loop(0, ZR)
    def _(r):
        for j in range(0, D, LANES):
            arow[r, pl.ds(j, LANES)] = jnp.zeros((LANES,), jnp.float32)

    row0 = sid * RPS
    _zero_acc_slice(arow, acc, row0)

    plsc.subcore_barrier()

    base = (cid * NS + sid) * EPW

    @pl.loop(0, NCH)
    def _(k):
        off = pl.multiple_of(base + k * CHUNK, 8)
        pltpu.sync_copy(src_hbm.at[pl.ds(off, CHUNK)], sidx)
        pltpu.sync_copy(dst_hbm.at[pl.ds(off, CHUNK)], didx)
        ca = pltpu.async_copy(a_hbm.at[didx], arow, sem0)
        cb = pltpu.async_copy(b_hbm.at[sidx], brow, sem1)
        ca.wait()
        cb.wait()

        @pl.loop(0, CHUNK)
        def _(e):
            vs = []
            for j in range(8):
                va = arow[e, pl.ds(j * LANES, LANES)]
                vb = brow[e, pl.ds(j * LANES, LANES)]
                vs.append(va + vb)
            s1 = ((vs[0] + vs[1]) + (vs[2] + vs[3])) + \
                 ((vs[4] + vs[5]) + (vs[6] + vs[7]))
            sq = [v * v for v in vs]
            s2 = ((sq[0] + sq[1]) + (sq[2] + sq[3])) + \
                 ((sq[4] + sq[5]) + (sq[6] + sq[7]))
            t1 = _lane_sum(s1)
            t2 = _lane_sum(s2)
            mu = t1 * (1.0 / 128.0)
            var = t2 * (1.0 / 128.0) - mu * mu + 1e-5
            bits = jax.lax.bitcast_convert_type(var, jnp.int32)
            y = jax.lax.bitcast_convert_type(
                jnp.int32(0x5F3759DF) - (bits >> 1), jnp.float32)
            for _ in range(3):
                y = y * (1.5 - (0.5 * var) * (y * y))
            for j in range(8):
                arow[e, pl.ds(j * LANES, LANES)] = \
                    jnp.maximum((vs[j] - mu) * y, 0.0)

        pltpu.sync_copy(arow, acc.at[didx], add=True)

    plsc.subcore_barrier()
    pltpu.sync_copy(acc.at[pl.ds(row0, RPS)], p_hbm.at[cid, pl.ds(row0, RPS)])


def _sc_mesh():
    return plsc.VectorSubcoreMesh(core_axis_name="c", subcore_axis_name="s",
                                  num_cores=NC, num_subcores=NS)


def _edge_pass(a, b, src, dst):
    scratch = [
        pltpu.VMEM((CHUNK,), jnp.int32),
        pltpu.VMEM((CHUNK,), jnp.int32),
        pltpu.VMEM((CHUNK, D), jnp.float32),
        pltpu.VMEM((CHUNK, D), jnp.float32),
        pltpu.VMEM_SHARED((NPAD, D), jnp.float32),
        pltpu.SemaphoreType.DMA,
        pltpu.SemaphoreType.DMA,
    ]
    fn = pl.kernel(_edge_body,
                   out_type=jax.ShapeDtypeStruct((NC, NPAD, D), jnp.float32),
                   mesh=_sc_mesh(), scratch_types=scratch)
    return fn(a, b, src, dst)


CCHUNK = 80
CNCH = EPW // CCHUNK


def _count_body(dst_hbm, c_hbm, didx, ones, cacc):
    cid = jax.lax.axis_index("c")
    sid = jax.lax.axis_index("s")

    @pl.loop(0, CCHUNK)
    def _(r):
        for j in range(0, D, LANES):
            ones[r, pl.ds(j, LANES)] = jnp.zeros((LANES,), jnp.float32)

    row0 = sid * RPS
    _zero_acc_slice(ones, cacc, row0)

    @pl.loop(0, CCHUNK)
    def _(r):
        for j in range(0, D, LANES):
            ones[r, pl.ds(j, LANES)] = jnp.ones((LANES,), jnp.float32)

    plsc.subcore_barrier()

    base = (cid * NS + sid) * EPW

    @pl.loop(0, CNCH)
    def _(k):
        off = pl.multiple_of(base + k * CCHUNK, 8)
        pltpu.sync_copy(dst_hbm.at[pl.ds(off, CCHUNK)], didx)
        pltpu.sync_copy(ones, cacc.at[didx], add=True)

    plsc.subcore_barrier()
    pltpu.sync_copy(cacc.at[pl.ds(row0, RPS)], c_hbm.at[cid, pl.ds(row0, RPS)])


def _count_pass(dst):
    scratch = [
        pltpu.VMEM((CCHUNK,), jnp.int32),
        pltpu.VMEM((CCHUNK, D), jnp.float32),
        pltpu.VMEM_SHARED((NPAD, D), jnp.float32),
    ]
    fn = pl.kernel(
        _count_body,
        out_type=jax.ShapeDtypeStruct((NC, NPAD, D), jnp.float32),
        mesh=_sc_mesh(), scratch_types=scratch)
    return fn(dst)


def kernel(x, edge_index, params):
    p = params
    h = _dense_call(_in_body, [D],
                    (x, 'row'), (p['in_W'], 'full'), (p['in_b'], 'full'),
                    (p['in_g'], 'full'), (p['in_beta'], 'full'))

    src = edge_index[0]
    dst = edge_index[1]

    cgath = _count_pass(dst)
    for i, lp in enumerate(p['layers']):
        wt = lp['msg_W'][:D]
        wb = lp['msg_W'][D:]
        a, b, sl = _dense_call(
            _ab_body, [D, D, D],
            (h, 'row'), (wt, 'full'), (wb, 'full'), (lp['msg_b'], 'full'),
            (lp['msg_g'], 'full'), (lp['msg_beta'], 'full'))

        psum = _edge_pass(a, b, src, dst)

        h = _dense_call(
            _upd_body_i(i > 0), [D],
            (psum, 'row3'), (sl, 'row'), (cgath, 'row3'), (h, 'row'),
            (lp['upd_W'][:D], 'full'), (lp['upd_W'][D:], 'full'),
            (lp['upd_b'], 'full'), (lp['upd_g'], 'full'),
            (lp['upd_beta'], 'full'))

    o = _dense_call(_out_body, [OUT],
                    (h, 'row'), (p['out_W1'], 'full'), (p['out_b1'], 'full'),
                    (p['out_g'], 'full'), (p['out_beta'], 'full'),
                    (p['out_W2'], 'full'), (p['out_b2'], 'full'))
    return o


# speedup vs baseline: 5.5254x; 1.3254x over previous
import functools

import jax
import jax.numpy as jnp
from jax.experimental import pallas as pl
from jax.experimental.pallas import tpu as pltpu
from jax.experimental.pallas import tpu_sc as plsc

N = 10000
E = 320000
D = 128
OUT = 64

ROWS = 1000
GRID = N // ROWS

NC = 2
NS = 16
LANES = 16

EPW = E // (NC * NS)
CHUNK = 80
NCH = EPW // CHUNK
NPAD = 10112
RPS = NPAD // NS
ZR = 64


def _ln(x, g, b):
    mu = jnp.mean(x, axis=-1, keepdims=True)
    var = jnp.mean((x - mu) ** 2, axis=-1, keepdims=True)
    return (x - mu) * jax.lax.rsqrt(var + 1e-5) * g + b


def _row_spec(cols):
    return pl.BlockSpec((ROWS, cols), lambda i: (i, 0))


def _row3_spec(cols):
    return pl.BlockSpec((NC, ROWS, cols), lambda i: (0, i, 0))


def _full_spec(shape):
    nd = len(shape)
    return pl.BlockSpec(shape, lambda i: (0,) * nd)


def _dense_call(body, out_cols_list, *args):
    in_specs = []
    for a, kind in args:
        if kind == 'row':
            in_specs.append(_row_spec(a.shape[-1]))
        elif kind == 'row3':
            in_specs.append(_row3_spec(a.shape[-1]))
        else:
            in_specs.append(_full_spec(a.shape))
    outs = [jax.ShapeDtypeStruct((N, c), jnp.float32) for c in out_cols_list]
    res = pl.pallas_call(
        body,
        grid=(GRID,),
        in_specs=in_specs,
        out_specs=[_row_spec(c) for c in out_cols_list],
        out_shape=outs,
    )(*[a for a, _ in args])
    return res[0] if len(out_cols_list) == 1 else res


def _in_body(x_ref, w_ref, b_ref, g_ref, beta_ref, o_ref):
    h = jnp.dot(x_ref[...], w_ref[...], preferred_element_type=jnp.float32)
    h = h + b_ref[...]
    o_ref[...] = jax.nn.relu(_ln(h, g_ref[...], beta_ref[...]))


def _ab_body(h_ref, wt_ref, wb_ref, mb_ref, g_ref, beta_ref,
             a_ref, b_ref, sl_ref):
    h = h_ref[...]
    a = jnp.dot(h, wt_ref[...], preferred_element_type=jnp.float32)
    b = jnp.dot(h, wb_ref[...], preferred_element_type=jnp.float32)
    b = b + mb_ref[...]
    a_ref[...] = a
    b_ref[...] = b
    sl_ref[...] = jax.nn.relu(_ln(a + b, g_ref[...], beta_ref[...]))


def _upd_body(residual, p_ref, sl_ref, c_ref, h_ref,
              ut_ref, ub_ref, bias_ref, g_ref, beta_ref, o_ref):
    h = h_ref[...]
    cnt = c_ref[0, :, :1] + c_ref[1, :, :1] + 1.0
    aggr = (p_ref[0] + p_ref[1] + sl_ref[...]) / cnt
    u = jnp.dot(aggr, ut_ref[...], preferred_element_type=jnp.float32)
    u = u + jnp.dot(h, ub_ref[...], preferred_element_type=jnp.float32)
    u = u + bias_ref[...]
    hn = jax.nn.relu(_ln(u, g_ref[...], beta_ref[...]))
    if residual:
        hn = 0.5 * (hn + h)
    o_ref[...] = hn


def _out_body(h_ref, w1_ref, b1_ref, g_ref, beta_ref, w2_ref, b2_ref, o_ref):
    o = jnp.dot(h_ref[...], w1_ref[...], preferred_element_type=jnp.float32)
    o = jax.nn.relu(_ln(o + b1_ref[...], g_ref[...], beta_ref[...]))
    o = jnp.dot(o, w2_ref[...], preferred_element_type=jnp.float32) + b2_ref[...]
    m = jnp.max(o, axis=-1, keepdims=True)
    lse = jnp.log(jnp.sum(jnp.exp(o - m), axis=-1, keepdims=True)) + m
    o_ref[...] = o - lse


@functools.cache
def _upd_body_i(residual):
    return functools.partial(_upd_body, residual)


def _lane_sum(v):
    dnums = jax.lax.GatherDimensionNumbers(
        offset_dims=(), collapsed_slice_dims=(0,), start_index_map=(0,))
    for k in (8, 4, 2, 1):
        idx = jax.lax.iota(jnp.int32, LANES) ^ k
        v = v + jax.lax.gather(
            v, idx[:, None], dnums, slice_sizes=(1,),
            mode=jax.lax.GatherScatterMode.PROMISE_IN_BOUNDS)
    return v

def _zero_acc_slice(zsrc, acc, row0):
    for k in range(RPS // ZR):
        pltpu.sync_copy(zsrc.at[pl.ds(0, ZR)],
                        acc.at[pl.ds(row0 + k * ZR, ZR)])
    pltpu.sync_copy(zsrc.at[pl.ds(0, RPS % ZR)],
                    acc.at[pl.ds(row0 + (RPS // ZR) * ZR, RPS % ZR)])


def _edge_body(a_hbm, b_hbm, src_hbm, dst_hbm, p_hbm,
               sidx0, didx0, arow0, brow0, sidx1, didx1, arow1, brow1,
               acc, sem0, sem1, sem2, sem3):
    cid = jax.lax.axis_index("c")
    sid = jax.lax.axis_index("s")

    @pl.loop(0, ZR)
    def _(r):
        for j in range(0, D, LANES):
            arow0[r, pl.ds(j, LANES)] = jnp.zeros((LANES,), jnp.float32)

    row0 = sid * RPS
    _zero_acc_slice(arow0, acc, row0)

    plsc.subcore_barrier()

    base = (cid * NS + sid) * EPW

    def start(k, sidx, didx, arow, brow, sa, sb):
        off = pl.multiple_of(base + k * CHUNK, 8)
        pltpu.sync_copy(src_hbm.at[pl.ds(off, CHUNK)], sidx)
        pltpu.sync_copy(dst_hbm.at[pl.ds(off, CHUNK)], didx)
        pltpu.async_copy(a_hbm.at[didx], arow, sa)
        pltpu.async_copy(b_hbm.at[sidx], brow, sb)

    def wait(sidx, didx, arow, brow, sa, sb):
        pltpu.make_async_copy(a_hbm.at[didx], arow, sa).wait()
        pltpu.make_async_copy(b_hbm.at[sidx], brow, sb).wait()

    def compute_scatter(didx, arow, brow):
        @pl.loop(0, CHUNK)
        def _(e):
            vs = []
            for j in range(8):
                va = arow[e, pl.ds(j * LANES, LANES)]
                vb = brow[e, pl.ds(j * LANES, LANES)]
                vs.append(va + vb)
            s1 = ((vs[0] + vs[1]) + (vs[2] + vs[3])) + \
                 ((vs[4] + vs[5]) + (vs[6] + vs[7]))
            sq = [v * v for v in vs]
            s2 = ((sq[0] + sq[1]) + (sq[2] + sq[3])) + \
                 ((sq[4] + sq[5]) + (sq[6] + sq[7]))
            t1 = _lane_sum(s1)
            t2 = _lane_sum(s2)
            mu = t1 * (1.0 / 128.0)
            var = t2 * (1.0 / 128.0) - mu * mu + 1e-5
            bits = jax.lax.bitcast_convert_type(var, jnp.int32)
            y = jax.lax.bitcast_convert_type(
                jnp.int32(0x5F3759DF) - (bits >> 1), jnp.float32)
            for _ in range(2):
                y = y * (1.5 - (0.5 * var) * (y * y))
            for j in range(8):
                arow[e, pl.ds(j * LANES, LANES)] = \
                    jnp.maximum((vs[j] - mu) * y, 0.0)

        pltpu.sync_copy(arow, acc.at[didx], add=True)

    set0 = (sidx0, didx0, arow0, brow0, sem0, sem1)
    set1 = (sidx1, didx1, arow1, brow1, sem2, sem3)

    start(0, *set0)

    @pl.loop(0, NCH // 2)
    def _(kk):
        k0 = 2 * kk
        start(k0 + 1, *set1)
        wait(*set0)
        compute_scatter(didx0, arow0, brow0)
        start(k0 + 2, *set0)
        wait(*set1)
        compute_scatter(didx1, arow1, brow1)

    wait(*set0)
    compute_scatter(didx0, arow0, brow0)

    plsc.subcore_barrier()
    pltpu.sync_copy(acc.at[pl.ds(row0, RPS)], p_hbm.at[cid, pl.ds(row0, RPS)])


def _sc_mesh():
    return plsc.VectorSubcoreMesh(core_axis_name="c", subcore_axis_name="s",
                                  num_cores=NC, num_subcores=NS)


def _edge_pass(a, b, src, dst):
    scratch = [
        pltpu.VMEM((CHUNK,), jnp.int32),
        pltpu.VMEM((CHUNK,), jnp.int32),
        pltpu.VMEM((CHUNK, D), jnp.float32),
        pltpu.VMEM((CHUNK, D), jnp.float32),
        pltpu.VMEM((CHUNK,), jnp.int32),
        pltpu.VMEM((CHUNK,), jnp.int32),
        pltpu.VMEM((CHUNK, D), jnp.float32),
        pltpu.VMEM((CHUNK, D), jnp.float32),
        pltpu.VMEM_SHARED((NPAD, D), jnp.float32),
        pltpu.SemaphoreType.DMA,
        pltpu.SemaphoreType.DMA,
        pltpu.SemaphoreType.DMA,
        pltpu.SemaphoreType.DMA,
    ]
    fn = pl.kernel(_edge_body,
                   out_type=jax.ShapeDtypeStruct((NC, NPAD, D), jnp.float32),
                   mesh=_sc_mesh(), scratch_types=scratch)
    return fn(a, b, src, dst)


CCHUNK = 80
CNCH = EPW // CCHUNK


def _count_body(dst_hbm, c_hbm, didx, ones, cacc):
    cid = jax.lax.axis_index("c")
    sid = jax.lax.axis_index("s")

    @pl.loop(0, CCHUNK)
    def _(r):
        for j in range(0, D, LANES):
            ones[r, pl.ds(j, LANES)] = jnp.zeros((LANES,), jnp.float32)

    row0 = sid * RPS
    _zero_acc_slice(ones, cacc, row0)

    @pl.loop(0, CCHUNK)
    def _(r):
        for j in range(0, D, LANES):
            ones[r, pl.ds(j, LANES)] = jnp.ones((LANES,), jnp.float32)

    plsc.subcore_barrier()

    base = (cid * NS + sid) * EPW

    @pl.loop(0, CNCH)
    def _(k):
        off = pl.multiple_of(base + k * CCHUNK, 8)
        pltpu.sync_copy(dst_hbm.at[pl.ds(off, CCHUNK)], didx)
        pltpu.sync_copy(ones, cacc.at[didx], add=True)

    plsc.subcore_barrier()
    pltpu.sync_copy(cacc.at[pl.ds(row0, RPS)], c_hbm.at[cid, pl.ds(row0, RPS)])


def _count_pass(dst):
    scratch = [
        pltpu.VMEM((CCHUNK,), jnp.int32),
        pltpu.VMEM((CCHUNK, D), jnp.float32),
        pltpu.VMEM_SHARED((NPAD, D), jnp.float32),
    ]
    fn = pl.kernel(
        _count_body,
        out_type=jax.ShapeDtypeStruct((NC, NPAD, D), jnp.float32),
        mesh=_sc_mesh(), scratch_types=scratch)
    return fn(dst)


def kernel(x, edge_index, params):
    p = params
    h = _dense_call(_in_body, [D],
                    (x, 'row'), (p['in_W'], 'full'), (p['in_b'], 'full'),
                    (p['in_g'], 'full'), (p['in_beta'], 'full'))

    src = edge_index[0]
    dst = edge_index[1]

    cgath = _count_pass(dst)
    for i, lp in enumerate(p['layers']):
        wt = lp['msg_W'][:D]
        wb = lp['msg_W'][D:]
        a, b, sl = _dense_call(
            _ab_body, [D, D, D],
            (h, 'row'), (wt, 'full'), (wb, 'full'), (lp['msg_b'], 'full'),
            (lp['msg_g'], 'full'), (lp['msg_beta'], 'full'))

        psum = _edge_pass(a, b, src, dst)

        h = _dense_call(
            _upd_body_i(i > 0), [D],
            (psum, 'row3'), (sl, 'row'), (cgath, 'row3'), (h, 'row'),
            (lp['upd_W'][:D], 'full'), (lp['upd_W'][D:], 'full'),
            (lp['upd_b'], 'full'), (lp['upd_g'], 'full'),
            (lp['upd_beta'], 'full'))

    o = _dense_call(_out_body, [OUT],
                    (h, 'row'), (p['out_W1'], 'full'), (p['out_b1'], 'full'),
                    (p['out_g'], 'full'), (p['out_beta'], 'full'),
                    (p['out_W2'], 'full'), (p['out_b2'], 'full'))
    return o


# speedup vs baseline: 6.2883x; 1.1381x over previous
import functools

import jax
import jax.numpy as jnp
from jax.experimental import pallas as pl
from jax.experimental.pallas import tpu as pltpu
from jax.experimental.pallas import tpu_sc as plsc

N = 10000
E = 320000
D = 128
OUT = 64

ROWS = 1000
GRID = N // ROWS

NC = 2
NS = 16
LANES = 16

EPW = E // (NC * NS)
CHUNK = 80
NCH = EPW // CHUNK
NPAD = 10112
RPS = NPAD // NS
ZR = 64


def _ln(x, g, b):
    mu = jnp.mean(x, axis=-1, keepdims=True)
    var = jnp.mean((x - mu) ** 2, axis=-1, keepdims=True)
    return (x - mu) * jax.lax.rsqrt(var + 1e-5) * g + b


def _row_spec(cols):
    return pl.BlockSpec((ROWS, cols), lambda i: (i, 0))


def _row3_spec(cols):
    return pl.BlockSpec((NC, ROWS, cols), lambda i: (0, i, 0))


def _full_spec(shape):
    nd = len(shape)
    return pl.BlockSpec(shape, lambda i: (0,) * nd)


def _dense_call(body, out_cols_list, *args):
    in_specs = []
    for a, kind in args:
        if kind == 'row':
            in_specs.append(_row_spec(a.shape[-1]))
        elif kind == 'row3':
            in_specs.append(_row3_spec(a.shape[-1]))
        else:
            in_specs.append(_full_spec(a.shape))
    outs = [jax.ShapeDtypeStruct((N, c), jnp.float32) for c in out_cols_list]
    res = pl.pallas_call(
        body,
        grid=(GRID,),
        in_specs=in_specs,
        out_specs=[_row_spec(c) for c in out_cols_list],
        out_shape=outs,
    )(*[a for a, _ in args])
    return res[0] if len(out_cols_list) == 1 else res


def _in_body(x_ref, w_ref, b_ref, g_ref, beta_ref, o_ref):
    h = jnp.dot(x_ref[...], w_ref[...], preferred_element_type=jnp.float32)
    h = h + b_ref[...]
    o_ref[...] = jax.nn.relu(_ln(h, g_ref[...], beta_ref[...]))


def _ab_body(h_ref, wt_ref, wb_ref, mb_ref, g_ref, beta_ref,
             a_ref, b_ref, sl_ref):
    h = h_ref[...]
    a = jnp.dot(h, wt_ref[...], preferred_element_type=jnp.float32)
    b = jnp.dot(h, wb_ref[...], preferred_element_type=jnp.float32)
    b = b + mb_ref[...]
    a_ref[...] = a
    b_ref[...] = b
    sl_ref[...] = jax.nn.relu(_ln(a + b, g_ref[...], beta_ref[...]))


def _upd_body(residual, p_ref, sl_ref, c_ref, h_ref,
              ut_ref, ub_ref, bias_ref, g_ref, beta_ref, o_ref):
    h = h_ref[...]
    cnt = c_ref[0, :, :1] + c_ref[1, :, :1] + 1.0
    aggr = (p_ref[0] + p_ref[1] + sl_ref[...]) / cnt
    u = jnp.dot(aggr, ut_ref[...], preferred_element_type=jnp.float32)
    u = u + jnp.dot(h, ub_ref[...], preferred_element_type=jnp.float32)
    u = u + bias_ref[...]
    hn = jax.nn.relu(_ln(u, g_ref[...], beta_ref[...]))
    if residual:
        hn = 0.5 * (hn + h)
    o_ref[...] = hn


def _out_body(h_ref, w1_ref, b1_ref, g_ref, beta_ref, w2_ref, b2_ref, o_ref):
    o = jnp.dot(h_ref[...], w1_ref[...], preferred_element_type=jnp.float32)
    o = jax.nn.relu(_ln(o + b1_ref[...], g_ref[...], beta_ref[...]))
    o = jnp.dot(o, w2_ref[...], preferred_element_type=jnp.float32) + b2_ref[...]
    m = jnp.max(o, axis=-1, keepdims=True)
    lse = jnp.log(jnp.sum(jnp.exp(o - m), axis=-1, keepdims=True)) + m
    o_ref[...] = o - lse


@functools.cache
def _upd_body_i(residual):
    return functools.partial(_upd_body, residual)


def _lane_sum(v):
    dnums = jax.lax.GatherDimensionNumbers(
        offset_dims=(), collapsed_slice_dims=(0,), start_index_map=(0,))
    for k in (8, 4, 2, 1):
        idx = jax.lax.iota(jnp.int32, LANES) ^ k
        v = v + jax.lax.gather(
            v, idx[:, None], dnums, slice_sizes=(1,),
            mode=jax.lax.GatherScatterMode.PROMISE_IN_BOUNDS)
    return v

def _zero_acc_slice(zsrc, acc, row0):
    for k in range(RPS // ZR):
        pltpu.sync_copy(zsrc.at[pl.ds(0, ZR)],
                        acc.at[pl.ds(row0 + k * ZR, ZR)])
    pltpu.sync_copy(zsrc.at[pl.ds(0, RPS % ZR)],
                    acc.at[pl.ds(row0 + (RPS // ZR) * ZR, RPS % ZR)])


def _edge_body(a_hbm, b_hbm, src_hbm, dst_hbm, p_hbm,
               sidx0, didx0, arow0, brow0, sidx1, didx1, arow1, brow1,
               acc, sem0, sem1, sem2, sem3):
    cid = jax.lax.axis_index("c")
    sid = jax.lax.axis_index("s")

    @pl.loop(0, ZR)
    def _(r):
        for j in range(0, D, LANES):
            arow0[r, pl.ds(j, LANES)] = jnp.zeros((LANES,), jnp.float32)

    row0 = sid * RPS
    _zero_acc_slice(arow0, acc, row0)

    plsc.subcore_barrier()

    base = (cid * NS + sid) * EPW

    def start(k, sidx, didx, arow, brow, sa, sb):
        off = pl.multiple_of(base + k * CHUNK, 8)
        pltpu.sync_copy(src_hbm.at[pl.ds(off, CHUNK)], sidx)
        pltpu.sync_copy(dst_hbm.at[pl.ds(off, CHUNK)], didx)
        pltpu.async_copy(a_hbm.at[didx], arow, sa)
        pltpu.async_copy(b_hbm.at[sidx], brow, sb)

    def wait(sidx, didx, arow, brow, sa, sb):
        pltpu.make_async_copy(a_hbm.at[didx], arow, sa).wait()
        pltpu.make_async_copy(b_hbm.at[sidx], brow, sb).wait()

    def compute_scatter(didx, arow, brow):
        @pl.loop(0, CHUNK // 2)
        def _(ee):
            e0 = 2 * ee
            e1 = e0 + 1
            vs0, vs1 = [], []
            for j in range(8):
                sl = pl.ds(j * LANES, LANES)
                vs0.append(arow[e0, sl] + brow[e0, sl])
                vs1.append(arow[e1, sl] + brow[e1, sl])

            def tree(v):
                return ((v[0] + v[1]) + (v[2] + v[3])) + \
                       ((v[4] + v[5]) + (v[6] + v[7]))

            s10 = tree(vs0)
            s11 = tree(vs1)
            s20 = tree([v * v for v in vs0])
            s21 = tree([v * v for v in vs1])
            t10 = _lane_sum(s10)
            t11 = _lane_sum(s11)
            t20 = _lane_sum(s20)
            t21 = _lane_sum(s21)

            def stats(t1, t2):
                mu = t1 * (1.0 / 128.0)
                var = t2 * (1.0 / 128.0) - mu * mu + 1e-5
                bits = jax.lax.bitcast_convert_type(var, jnp.int32)
                y = jax.lax.bitcast_convert_type(
                    jnp.int32(0x5F3759DF) - (bits >> 1), jnp.float32)
                for _ in range(2):
                    y = y * (1.5 - (0.5 * var) * (y * y))
                return mu, y

            mu0, y0 = stats(t10, t20)
            mu1, y1 = stats(t11, t21)
            for j in range(8):
                sl = pl.ds(j * LANES, LANES)
                arow[e0, sl] = jnp.maximum((vs0[j] - mu0) * y0, 0.0)
                arow[e1, sl] = jnp.maximum((vs1[j] - mu1) * y1, 0.0)

        pltpu.sync_copy(arow, acc.at[didx], add=True)

    set0 = (sidx0, didx0, arow0, brow0, sem0, sem1)
    set1 = (sidx1, didx1, arow1, brow1, sem2, sem3)

    start(0, *set0)

    @pl.loop(0, NCH // 2)
    def _(kk):
        k0 = 2 * kk
        start(k0 + 1, *set1)
        wait(*set0)
        compute_scatter(didx0, arow0, brow0)
        start(k0 + 2, *set0)
        wait(*set1)
        compute_scatter(didx1, arow1, brow1)

    wait(*set0)
    compute_scatter(didx0, arow0, brow0)

    plsc.subcore_barrier()
    pltpu.sync_copy(acc.at[pl.ds(row0, RPS)], p_hbm.at[cid, pl.ds(row0, RPS)])


def _sc_mesh():
    return plsc.VectorSubcoreMesh(core_axis_name="c", subcore_axis_name="s",
                                  num_cores=NC, num_subcores=NS)


def _edge_pass(a, b, src, dst):
    scratch = [
        pltpu.VMEM((CHUNK,), jnp.int32),
        pltpu.VMEM((CHUNK,), jnp.int32),
        pltpu.VMEM((CHUNK, D), jnp.float32),
        pltpu.VMEM((CHUNK, D), jnp.float32),
        pltpu.VMEM((CHUNK,), jnp.int32),
        pltpu.VMEM((CHUNK,), jnp.int32),
        pltpu.VMEM((CHUNK, D), jnp.float32),
        pltpu.VMEM((CHUNK, D), jnp.float32),
        pltpu.VMEM_SHARED((NPAD, D), jnp.float32),
        pltpu.SemaphoreType.DMA,
        pltpu.SemaphoreType.DMA,
        pltpu.SemaphoreType.DMA,
        pltpu.SemaphoreType.DMA,
    ]
    fn = pl.kernel(_edge_body,
                   out_type=jax.ShapeDtypeStruct((NC, NPAD, D), jnp.float32),
                   mesh=_sc_mesh(), scratch_types=scratch)
    return fn(a, b, src, dst)


CCHUNK = 80
CNCH = EPW // CCHUNK


def _count_body(dst_hbm, c_hbm, didx, ones, cacc):
    cid = jax.lax.axis_index("c")
    sid = jax.lax.axis_index("s")

    @pl.loop(0, CCHUNK)
    def _(r):
        for j in range(0, D, LANES):
            ones[r, pl.ds(j, LANES)] = jnp.zeros((LANES,), jnp.float32)

    row0 = sid * RPS
    _zero_acc_slice(ones, cacc, row0)

    @pl.loop(0, CCHUNK)
    def _(r):
        for j in range(0, D, LANES):
            ones[r, pl.ds(j, LANES)] = jnp.ones((LANES,), jnp.float32)

    plsc.subcore_barrier()

    base = (cid * NS + sid) * EPW

    @pl.loop(0, CNCH)
    def _(k):
        off = pl.multiple_of(base + k * CCHUNK, 8)
        pltpu.sync_copy(dst_hbm.at[pl.ds(off, CCHUNK)], didx)
        pltpu.sync_copy(ones, cacc.at[didx], add=True)

    plsc.subcore_barrier()
    pltpu.sync_copy(cacc.at[pl.ds(row0, RPS)], c_hbm.at[cid, pl.ds(row0, RPS)])


def _count_pass(dst):
    scratch = [
        pltpu.VMEM((CCHUNK,), jnp.int32),
        pltpu.VMEM((CCHUNK, D), jnp.float32),
        pltpu.VMEM_SHARED((NPAD, D), jnp.float32),
    ]
    fn = pl.kernel(
        _count_body,
        out_type=jax.ShapeDtypeStruct((NC, NPAD, D), jnp.float32),
        mesh=_sc_mesh(), scratch_types=scratch)
    return fn(dst)


def kernel(x, edge_index, params):
    p = params
    h = _dense_call(_in_body, [D],
                    (x, 'row'), (p['in_W'], 'full'), (p['in_b'], 'full'),
                    (p['in_g'], 'full'), (p['in_beta'], 'full'))

    src = edge_index[0]
    dst = edge_index[1]

    cgath = _count_pass(dst)
    for i, lp in enumerate(p['layers']):
        wt = lp['msg_W'][:D]
        wb = lp['msg_W'][D:]
        a, b, sl = _dense_call(
            _ab_body, [D, D, D],
            (h, 'row'), (wt, 'full'), (wb, 'full'), (lp['msg_b'], 'full'),
            (lp['msg_g'], 'full'), (lp['msg_beta'], 'full'))

        psum = _edge_pass(a, b, src, dst)

        h = _dense_call(
            _upd_body_i(i > 0), [D],
            (psum, 'row3'), (sl, 'row'), (cgath, 'row3'), (h, 'row'),
            (lp['upd_W'][:D], 'full'), (lp['upd_W'][D:], 'full'),
            (lp['upd_b'], 'full'), (lp['upd_g'], 'full'),
            (lp['upd_beta'], 'full'))

    o = _dense_call(_out_body, [OUT],
                    (h, 'row'), (p['out_W1'], 'full'), (p['out_b1'], 'full'),
                    (p['out_g'], 'full'), (p['out_beta'], 'full'),
                    (p['out_W2'], 'full'), (p['out_b2'], 'full'))
    return o


# speedup vs baseline: 7.1790x; 1.1416x over previous
import functools

import jax
import jax.numpy as jnp
from jax.experimental import pallas as pl
from jax.experimental.pallas import tpu as pltpu
from jax.experimental.pallas import tpu_sc as plsc

N = 10000
E = 320000
D = 128
OUT = 64

ROWS = 1000
GRID = N // ROWS

NC = 2
NS = 16
LANES = 16

EPW = E // (NC * NS)
CHUNK = 80
NCH = EPW // CHUNK
NPAD = 10112
RPS = NPAD // NS
ZR = 64


def _ln(x, g, b):
    mu = jnp.mean(x, axis=-1, keepdims=True)
    var = jnp.mean((x - mu) ** 2, axis=-1, keepdims=True)
    return (x - mu) * jax.lax.rsqrt(var + 1e-5) * g + b


def _row_spec(cols):
    return pl.BlockSpec((ROWS, cols), lambda i: (i, 0))


def _row3_spec(cols):
    return pl.BlockSpec((NC, ROWS, cols), lambda i: (0, i, 0))


def _full_spec(shape):
    nd = len(shape)
    return pl.BlockSpec(shape, lambda i: (0,) * nd)


def _dense_call(body, out_cols_list, *args):
    in_specs = []
    for a, kind in args:
        if kind == 'row':
            in_specs.append(_row_spec(a.shape[-1]))
        elif kind == 'row3':
            in_specs.append(_row3_spec(a.shape[-1]))
        else:
            in_specs.append(_full_spec(a.shape))
    outs = [jax.ShapeDtypeStruct((N, c), jnp.float32) for c in out_cols_list]
    res = pl.pallas_call(
        body,
        grid=(GRID,),
        in_specs=in_specs,
        out_specs=[_row_spec(c) for c in out_cols_list],
        out_shape=outs,
    )(*[a for a, _ in args])
    return res[0] if len(out_cols_list) == 1 else res


def _in_body(x_ref, w_ref, b_ref, g_ref, beta_ref, o_ref):
    h = jnp.dot(x_ref[...], w_ref[...], preferred_element_type=jnp.float32)
    h = h + b_ref[...]
    o_ref[...] = jax.nn.relu(_ln(h, g_ref[...], beta_ref[...]))


def _ab_body(h_ref, wt_ref, wb_ref, mb_ref, g_ref, beta_ref,
             a_ref, b_ref, sl_ref):
    h = h_ref[...]
    a = jnp.dot(h, wt_ref[...], preferred_element_type=jnp.float32)
    b = jnp.dot(h, wb_ref[...], preferred_element_type=jnp.float32)
    b = b + mb_ref[...]
    a_ref[...] = a
    b_ref[...] = b
    sl_ref[...] = jax.nn.relu(_ln(a + b, g_ref[...], beta_ref[...]))


def _upd_body(residual, p_ref, sl_ref, c_ref, h_ref,
              ut_ref, ub_ref, bias_ref, g_ref, beta_ref, o_ref):
    h = h_ref[...]
    cnt = c_ref[0, :, :1] + c_ref[1, :, :1] + 1.0
    aggr = (p_ref[0] + p_ref[1] + sl_ref[...]) / cnt
    u = jnp.dot(aggr, ut_ref[...], preferred_element_type=jnp.float32)
    u = u + jnp.dot(h, ub_ref[...], preferred_element_type=jnp.float32)
    u = u + bias_ref[...]
    hn = jax.nn.relu(_ln(u, g_ref[...], beta_ref[...]))
    if residual:
        hn = 0.5 * (hn + h)
    o_ref[...] = hn


def _out_body(h_ref, w1_ref, b1_ref, g_ref, beta_ref, w2_ref, b2_ref, o_ref):
    o = jnp.dot(h_ref[...], w1_ref[...], preferred_element_type=jnp.float32)
    o = jax.nn.relu(_ln(o + b1_ref[...], g_ref[...], beta_ref[...]))
    o = jnp.dot(o, w2_ref[...], preferred_element_type=jnp.float32) + b2_ref[...]
    m = jnp.max(o, axis=-1, keepdims=True)
    lse = jnp.log(jnp.sum(jnp.exp(o - m), axis=-1, keepdims=True)) + m
    o_ref[...] = o - lse


@functools.cache
def _upd_body_i(residual):
    return functools.partial(_upd_body, residual)


def _lane_sum(v):
    dnums = jax.lax.GatherDimensionNumbers(
        offset_dims=(), collapsed_slice_dims=(0,), start_index_map=(0,))
    for k in (8, 4, 2, 1):
        idx = jax.lax.iota(jnp.int32, LANES) ^ k
        v = v + jax.lax.gather(
            v, idx[:, None], dnums, slice_sizes=(1,),
            mode=jax.lax.GatherScatterMode.PROMISE_IN_BOUNDS)
    return v

def _zero_acc_slice(zsrc, acc, row0):
    for k in range(RPS // ZR):
        pltpu.sync_copy(zsrc.at[pl.ds(0, ZR)],
                        acc.at[pl.ds(row0 + k * ZR, ZR)])
    pltpu.sync_copy(zsrc.at[pl.ds(0, RPS % ZR)],
                    acc.at[pl.ds(row0 + (RPS // ZR) * ZR, RPS % ZR)])


def _edge_body(a_hbm, b_hbm, src_hbm, dst_hbm, p_hbm,
               sidx0, didx0, arow0, brow0, sidx1, didx1, arow1, brow1,
               acc, sem0, sem1, sem2, sem3):
    cid = jax.lax.axis_index("c")
    sid = jax.lax.axis_index("s")

    @pl.loop(0, ZR)
    def _(r):
        for j in range(0, D, LANES):
            arow0[r, pl.ds(j, LANES)] = jnp.zeros((LANES,), jnp.float32)

    row0 = sid * RPS
    _zero_acc_slice(arow0, acc, row0)

    plsc.subcore_barrier()

    base = (cid * NS + sid) * EPW

    def start(k, sidx, didx, arow, brow, sa, sb):
        off = pl.multiple_of(base + k * CHUNK, 8)
        pltpu.sync_copy(src_hbm.at[pl.ds(off, CHUNK)], sidx)
        pltpu.sync_copy(dst_hbm.at[pl.ds(off, CHUNK)], didx)
        pltpu.async_copy(a_hbm.at[didx], arow, sa)
        pltpu.async_copy(b_hbm.at[sidx], brow, sb)

    def wait(sidx, didx, arow, brow, sa, sb):
        pltpu.make_async_copy(a_hbm.at[didx], arow, sa).wait()
        pltpu.make_async_copy(b_hbm.at[sidx], brow, sb).wait()

    def compute_scatter(didx, arow, brow):
        U = 4

        @pl.loop(0, CHUNK // U)
        def _(ee):
            es = [U * ee + u for u in range(U)]
            vss = [[] for _ in range(U)]
            for j in range(8):
                sl = pl.ds(j * LANES, LANES)
                for u, e in enumerate(es):
                    vss[u].append(arow[e, sl] + brow[e, sl])

            def tree(v):
                return ((v[0] + v[1]) + (v[2] + v[3])) + \
                       ((v[4] + v[5]) + (v[6] + v[7]))

            t1s = [_lane_sum(tree(vs)) for vs in vss]
            t2s = [_lane_sum(tree([v * v for v in vs])) for vs in vss]

            def stats(t1, t2):
                mu = t1 * (1.0 / 128.0)
                var = t2 * (1.0 / 128.0) - mu * mu + 1e-5
                bits = jax.lax.bitcast_convert_type(var, jnp.int32)
                y = jax.lax.bitcast_convert_type(
                    jnp.int32(0x5F3759DF) - (bits >> 1), jnp.float32)
                for _ in range(2):
                    y = y * (1.5 - (0.5 * var) * (y * y))
                return mu, y

            ms = [stats(t1, t2) for t1, t2 in zip(t1s, t2s)]
            for j in range(8):
                sl = pl.ds(j * LANES, LANES)
                for u, e in enumerate(es):
                    mu, y = ms[u]
                    arow[e, sl] = jnp.maximum((vss[u][j] - mu) * y, 0.0)

        pltpu.sync_copy(arow, acc.at[didx], add=True)

    set0 = (sidx0, didx0, arow0, brow0, sem0, sem1)
    set1 = (sidx1, didx1, arow1, brow1, sem2, sem3)

    start(0, *set0)

    @pl.loop(0, NCH // 2)
    def _(kk):
        k0 = 2 * kk
        start(k0 + 1, *set1)
        wait(*set0)
        compute_scatter(didx0, arow0, brow0)
        start(k0 + 2, *set0)
        wait(*set1)
        compute_scatter(didx1, arow1, brow1)

    wait(*set0)
    compute_scatter(didx0, arow0, brow0)

    plsc.subcore_barrier()
    pltpu.sync_copy(acc.at[pl.ds(row0, RPS)], p_hbm.at[cid, pl.ds(row0, RPS)])


def _sc_mesh():
    return plsc.VectorSubcoreMesh(core_axis_name="c", subcore_axis_name="s",
                                  num_cores=NC, num_subcores=NS)


def _edge_pass(a, b, src, dst):
    scratch = [
        pltpu.VMEM((CHUNK,), jnp.int32),
        pltpu.VMEM((CHUNK,), jnp.int32),
        pltpu.VMEM((CHUNK, D), jnp.float32),
        pltpu.VMEM((CHUNK, D), jnp.float32),
        pltpu.VMEM((CHUNK,), jnp.int32),
        pltpu.VMEM((CHUNK,), jnp.int32),
        pltpu.VMEM((CHUNK, D), jnp.float32),
        pltpu.VMEM((CHUNK, D), jnp.float32),
        pltpu.VMEM_SHARED((NPAD, D), jnp.float32),
        pltpu.SemaphoreType.DMA,
        pltpu.SemaphoreType.DMA,
        pltpu.SemaphoreType.DMA,
        pltpu.SemaphoreType.DMA,
    ]
    fn = pl.kernel(_edge_body,
                   out_type=jax.ShapeDtypeStruct((NC, NPAD, D), jnp.float32),
                   mesh=_sc_mesh(), scratch_types=scratch)
    return fn(a, b, src, dst)


CCHUNK = 80
CNCH = EPW // CCHUNK


def _count_body(dst_hbm, c_hbm, didx, ones, cacc):
    cid = jax.lax.axis_index("c")
    sid = jax.lax.axis_index("s")

    @pl.loop(0, CCHUNK)
    def _(r):
        for j in range(0, D, LANES):
            ones[r, pl.ds(j, LANES)] = jnp.zeros((LANES,), jnp.float32)

    row0 = sid * RPS
    _zero_acc_slice(ones, cacc, row0)

    @pl.loop(0, CCHUNK)
    def _(r):
        for j in range(0, D, LANES):
            ones[r, pl.ds(j, LANES)] = jnp.ones((LANES,), jnp.float32)

    plsc.subcore_barrier()

    base = (cid * NS + sid) * EPW

    @pl.loop(0, CNCH)
    def _(k):
        off = pl.multiple_of(base + k * CCHUNK, 8)
        pltpu.sync_copy(dst_hbm.at[pl.ds(off, CCHUNK)], didx)
        pltpu.sync_copy(ones, cacc.at[didx], add=True)

    plsc.subcore_barrier()
    pltpu.sync_copy(cacc.at[pl.ds(row0, RPS)], c_hbm.at[cid, pl.ds(row0, RPS)])


def _count_pass(dst):
    scratch = [
        pltpu.VMEM((CCHUNK,), jnp.int32),
        pltpu.VMEM((CCHUNK, D), jnp.float32),
        pltpu.VMEM_SHARED((NPAD, D), jnp.float32),
    ]
    fn = pl.kernel(
        _count_body,
        out_type=jax.ShapeDtypeStruct((NC, NPAD, D), jnp.float32),
        mesh=_sc_mesh(), scratch_types=scratch)
    return fn(dst)


def kernel(x, edge_index, params):
    p = params
    h = _dense_call(_in_body, [D],
                    (x, 'row'), (p['in_W'], 'full'), (p['in_b'], 'full'),
                    (p['in_g'], 'full'), (p['in_beta'], 'full'))

    src = edge_index[0]
    dst = edge_index[1]

    cgath = _count_pass(dst)
    for i, lp in enumerate(p['layers']):
        wt = lp['msg_W'][:D]
        wb = lp['msg_W'][D:]
        a, b, sl = _dense_call(
            _ab_body, [D, D, D],
            (h, 'row'), (wt, 'full'), (wb, 'full'), (lp['msg_b'], 'full'),
            (lp['msg_g'], 'full'), (lp['msg_beta'], 'full'))

        psum = _edge_pass(a, b, src, dst)

        h = _dense_call(
            _upd_body_i(i > 0), [D],
            (psum, 'row3'), (sl, 'row'), (cgath, 'row3'), (h, 'row'),
            (lp['upd_W'][:D], 'full'), (lp['upd_W'][D:], 'full'),
            (lp['upd_b'], 'full'), (lp['upd_g'], 'full'),
            (lp['upd_beta'], 'full'))

    o = _dense_call(_out_body, [OUT],
                    (h, 'row'), (p['out_W1'], 'full'), (p['out_b1'], 'full'),
                    (p['out_g'], 'full'), (p['out_beta'], 'full'),
                    (p['out_W2'], 'full'), (p['out_b2'], 'full'))
    return o
